# Initial kernel scaffold; baseline (speedup 1.0000x reference)
import jax
import jax.numpy as jnp
from jax.experimental import pallas as pl


def kernel(x, W):
    raise NotImplementedError("write your pallas kernel here")


# speedup vs baseline: 23.7416x; 23.7416x over previous
import functools

import jax
import jax.numpy as jnp
from jax import lax
from jax.experimental import pallas as pl
from jax.experimental.pallas import tpu as pltpu

_K = 15
_BR = 256
_NEG = -3.4e38


def _topk_matmul_kernel(x_ref, wt_ref, o_ref):
    x = x_ref[...]
    col = lax.broadcasted_iota(jnp.int32, x.shape, 1)
    mask = jnp.zeros(x.shape, jnp.float32)
    for _ in range(_K):
        m = jnp.max(x, axis=1, keepdims=True)
        first = jnp.min(jnp.where(x == m, col, jnp.int32(1 << 30)), axis=1,
                        keepdims=True)
        hit = col == first
        mask = jnp.where(hit, 1.0, mask)
        x = jnp.where(hit, _NEG, x)
    o_ref[...] = jax.lax.dot(
        mask.astype(jnp.bfloat16), wt_ref[...],
        precision=lax.Precision.DEFAULT,
        preferred_element_type=jnp.float32)


def kernel(x, W):
    batch, qd = x.shape
    ed = W.shape[0]
    wt = W.T.astype(jnp.bfloat16)
    grid = (batch // _BR,)
    out = pl.pallas_call(
        _topk_matmul_kernel,
        grid=grid,
        in_specs=[
            pl.BlockSpec((_BR, qd), lambda i: (i, 0)),
            pl.BlockSpec((qd, ed), lambda i: (0, 0)),
        ],
        out_specs=pl.BlockSpec((_BR, ed), lambda i: (i, 0)),
        out_shape=jax.ShapeDtypeStruct((batch, ed), jnp.float32),
    )(x, wt)
    return out
